# baseline (speedup 1.0000x reference)
import functools

import jax
import jax.numpy as jnp
from jax import lax
from jax.experimental import pallas as pl
from jax.experimental.pallas import tpu as pltpu
from jax.experimental.pallas import tpu_sc as plsc

N_NODES = 100000
MEM_DIM = 128
MSG_DIM = 256
B = 4096

_NC = 2
_NS = 16
_NW = _NC * _NS
_CHUNK = B // _NW

_CP_CHUNK = 3200
_CP_LAST = N_NODES - _CP_CHUNK * (_NW - 1)


def _sc_mesh():
    return plsc.VectorSubcoreMesh(
        core_axis_name="c", subcore_axis_name="s", num_cores=_NC, num_subcores=_NS
    )


def _worker_id():
    return lax.axis_index("s") * _NC + lax.axis_index("c")


def _sc_gather(mem, idx):
    @functools.partial(
        pl.kernel,
        out_type=jax.ShapeDtypeStruct((B, MEM_DIM), jnp.float32),
        mesh=_sc_mesh(),
        scratch_types=[
            pltpu.VMEM((_CHUNK,), jnp.int32),
            pltpu.VMEM((_CHUNK, MEM_DIM), jnp.float32),
            pltpu.SemaphoreType.DMA,
        ],
    )
    def gk(mem_hbm, idx_hbm, out_hbm, idx_v, rows_v, sem):
        base = _worker_id() * _CHUNK
        pltpu.sync_copy(idx_hbm.at[pl.ds(base, _CHUNK)], idx_v)
        pltpu.async_copy(mem_hbm.at[idx_v], rows_v, sem).wait()
        pltpu.sync_copy(rows_v, out_hbm.at[pl.ds(base, _CHUNK)])

    return gk(mem, idx)


_GRU_BLK = 1024
_JL_CHUNK = 512
_JL_IBLK = 1024


_CPB = 4000
_NCH = N_NODES // _CPB
_CPS = 6


def _comp_body(x_ref, h_ref, wih_ref, whh_ref, bih_ref, bhh_ref,
               nlane_ref, ncol_ref, mem_hbm,
               newh_ref, jl_ref, memout_hbm,
               nb_scratch, *bufs_and_sems):
    bufs = bufs_and_sems[:7]
    isem = bufs_and_sems[7:14]
    osem = bufs_and_sems[14:21]
    i = pl.program_id(0)

    def chunk_at(ref, g):
        off = pl.multiple_of(g * _CPB, _CPB)
        return ref.at[pl.ds(off, _CPB)]

    for kk in range(_CPS):
        @pl.when(i > 0)
        def _drain(kk=kk):
            pltpu.make_async_copy(bufs[kk], chunk_at(memout_hbm, 0), osem[kk]).wait()
        g = i * _CPS + kk
        pltpu.make_async_copy(chunk_at(mem_hbm, g), bufs[kk], isem[kk]).start()

    @pl.when(i == B // _GRU_BLK - 1)
    def _extra_in():
        pltpu.make_async_copy(chunk_at(mem_hbm, _NCH - 1), bufs[6], isem[6]).start()

    @pl.when(i == 0)
    def _build_nbcast():
        nb_scratch[...] = jnp.broadcast_to(ncol_ref[...], (B, 128))

    x = x_ref[...].astype(jnp.bfloat16)
    h32 = h_ref[...]
    h = h32.astype(jnp.bfloat16)
    dn = (((1,), (1,)), ((), ()))
    wih = wih_ref[...].astype(jnp.bfloat16)
    whh = whh_ref[...].astype(jnp.bfloat16)
    gi = lax.dot_general(x, wih, dn, preferred_element_type=jnp.float32) + bih_ref[...]
    gh = lax.dot_general(h, whh, dn, preferred_element_type=jnp.float32) + bhh_ref[...]
    i_r, i_z, i_n = gi[:, :MEM_DIM], gi[:, MEM_DIM : 2 * MEM_DIM], gi[:, 2 * MEM_DIM :]
    h_r, h_z, h_n = gh[:, :MEM_DIM], gh[:, MEM_DIM : 2 * MEM_DIM], gh[:, 2 * MEM_DIM :]
    r = jax.nn.sigmoid(i_r + h_r)
    z = jax.nn.sigmoid(i_z + h_z)
    n = jnp.tanh(i_n + r * h_n)
    newh_ref[...] = n + z * (h32 - n)

    for kk in range(_CPS):
        g = i * _CPS + kk
        pltpu.make_async_copy(chunk_at(mem_hbm, g), bufs[kk], isem[kk]).wait()
        pltpu.make_async_copy(bufs[kk], chunk_at(memout_hbm, g), osem[kk]).start()

    ni = nlane_ref[0]
    nrows = _JL_IBLK // 128
    nchunks = B // _JL_CHUNK

    def _sweep(start):
        def go(ni_op):
            bests = [jnp.full((1, 128), -1, jnp.int32) for _ in range(nrows)]
            for c in range(start, nchunks):
                nj = nb_scratch[pl.ds(c * _JL_CHUNK, _JL_CHUNK), :]
                jv = lax.broadcasted_iota(jnp.int32, (_JL_CHUNK, 128), 0) + c * _JL_CHUNK
                for rr in range(nrows):
                    m = jnp.where(nj == ni_op[rr : rr + 1, :], jv, -1)
                    bests[rr] = jnp.maximum(bests[rr], jnp.max(m, axis=0, keepdims=True))
            return jnp.concatenate(bests, axis=0)
        return go

    nsweep = _JL_IBLK // _JL_CHUNK
    jl_ref[0] = lax.switch(i, [_sweep(s * nsweep) for s in range(B // _JL_IBLK)], ni)

    @pl.when(i == B // _GRU_BLK - 1)
    def _final_drain():
        pltpu.make_async_copy(chunk_at(mem_hbm, _NCH - 1), bufs[6], isem[6]).wait()
        pltpu.make_async_copy(bufs[6], chunk_at(memout_hbm, _NCH - 1), osem[6]).start()
        for kk in range(_CPS):
            pltpu.make_async_copy(bufs[kk], chunk_at(memout_hbm, 0), osem[kk]).wait()
        pltpu.make_async_copy(bufs[6], chunk_at(memout_hbm, 0), osem[6]).wait()


def _tc_compute(x, h, W_ih, W_hh, b_ih, b_hh, nodes, mem):
    bih = b_ih.reshape(1, -1)
    bhh = b_hh.reshape(1, -1)
    nlane = nodes.reshape(B // _JL_IBLK, _JL_IBLK // 128, 128)
    ncol = nodes.reshape(B, 1)
    blk = lambda i: (i, 0)
    blk3 = lambda i: (i, 0, 0)
    const2 = lambda i: (0, 0)
    new_h, jl, mem_out = pl.pallas_call(
        _comp_body,
        grid=(B // _GRU_BLK,),
        in_specs=[
            pl.BlockSpec((_GRU_BLK, MSG_DIM), blk),
            pl.BlockSpec((_GRU_BLK, MEM_DIM), blk),
            pl.BlockSpec((3 * MEM_DIM, MSG_DIM), const2),
            pl.BlockSpec((3 * MEM_DIM, MEM_DIM), const2),
            pl.BlockSpec((1, 3 * MEM_DIM), const2),
            pl.BlockSpec((1, 3 * MEM_DIM), const2),
            pl.BlockSpec((1, _JL_IBLK // 128, 128), blk3),
            pl.BlockSpec((B, 1), const2),
            pl.BlockSpec(memory_space=pltpu.HBM),
        ],
        out_specs=[
            pl.BlockSpec((_GRU_BLK, MEM_DIM), blk),
            pl.BlockSpec((1, _JL_IBLK // 128, 128), blk3),
            pl.BlockSpec(memory_space=pltpu.HBM),
        ],
        out_shape=[
            jax.ShapeDtypeStruct((B, MEM_DIM), jnp.float32),
            jax.ShapeDtypeStruct((B // _JL_IBLK, _JL_IBLK // 128, 128), jnp.int32),
            jax.ShapeDtypeStruct((N_NODES, MEM_DIM), jnp.float32),
        ],
        scratch_shapes=[pltpu.VMEM((B, 128), jnp.int32)]
        + [pltpu.VMEM((_CPB, MEM_DIM), jnp.float32) for _ in range(7)]
        + [pltpu.SemaphoreType.DMA for _ in range(14)],
    )(x, h, W_ih, W_hh, bih, bhh, nlane, ncol, mem)
    return new_h, jl, mem_out


def _sc_scatter(new_h, j_last, idx, ts, mem_ref, lu_ref):
    @functools.partial(
        pl.kernel,
        out_type=(),
        mesh=_sc_mesh(),
        scratch_types=[
            pltpu.VMEM((_CHUNK // 2,), jnp.int32),
            pltpu.VMEM((_CHUNK // 2,), jnp.int32),
            pltpu.VMEM((_CHUNK // 2,), jnp.int32),
            pltpu.VMEM((_CHUNK // 2,), jnp.int32),
            pltpu.VMEM((_CHUNK // 2, MEM_DIM), jnp.float32),
            pltpu.VMEM((_CHUNK // 2, MEM_DIM), jnp.float32),
            pltpu.VMEM((_CHUNK,), jnp.float32),
            pltpu.SemaphoreType.DMA,
            pltpu.SemaphoreType.DMA,
            pltpu.SemaphoreType.DMA,
        ],
    )
    def sk(newh_hbm, jl_hbm, idx_hbm, ts_hbm, outmem_hbm, outlu_hbm,
           jla_v, jlb_v, idxa_v, idxb_v, rowsa_v, rowsb_v, ts_v,
           sema, semb, semt):
        w = _worker_id()
        base = w * _CHUNK
        half = _CHUNK // 2
        pltpu.sync_copy(jl_hbm.at[w // 8, w % 8, pl.ds(0, half)], jla_v)
        pltpu.sync_copy(jl_hbm.at[w // 8, w % 8, pl.ds(half, half)], jlb_v)
        pltpu.sync_copy(idx_hbm.at[pl.ds(base, half)], idxa_v)
        pltpu.sync_copy(idx_hbm.at[pl.ds(base + half, half)], idxb_v)
        ga = pltpu.async_copy(newh_hbm.at[jla_v], rowsa_v, sema)
        gb = pltpu.async_copy(newh_hbm.at[jlb_v], rowsb_v, semb)
        gt = pltpu.async_copy(ts_hbm.at[jla_v], ts_v.at[pl.ds(0, half)], semt)
        ga.wait()
        sa = pltpu.async_copy(rowsa_v, outmem_hbm.at[idxa_v], sema)
        gb.wait()
        sb = pltpu.async_copy(rowsb_v, outmem_hbm.at[idxb_v], semb)
        gt.wait()
        gt2 = pltpu.async_copy(ts_hbm.at[jlb_v], ts_v.at[pl.ds(half, half)], semt)
        gt2.wait()
        st1 = pltpu.async_copy(ts_v.at[pl.ds(0, half)], outlu_hbm.at[idxa_v], semt)
        st1.wait()
        st2 = pltpu.async_copy(ts_v.at[pl.ds(half, half)], outlu_hbm.at[idxb_v], semt)
        sa.wait()
        sb.wait()
        st2.wait()

    sk(new_h, j_last, idx, ts, mem_ref, lu_ref)


def kernel(memory_tensor, last_update, unique_nodes, unique_messages, unique_ts, W_ih, W_hh, b_ih, b_hh):
    h = _sc_gather(memory_tensor, unique_nodes)
    new_h, j_last, mem_out = _tc_compute(
        unique_messages, h, W_ih, W_hh, b_ih, b_hh, unique_nodes, memory_tensor
    )
    mem_ref = jax.new_ref(mem_out)
    lu_ref = jax.new_ref(last_update)
    _sc_scatter(new_h, j_last, unique_nodes, unique_ts, mem_ref, lu_ref)
    return mem_ref[...], lu_ref[...]


# speedup vs baseline: 1.0303x; 1.0303x over previous
import functools

import jax
import jax.numpy as jnp
from jax import lax
from jax.experimental import pallas as pl
from jax.experimental.pallas import tpu as pltpu
from jax.experimental.pallas import tpu_sc as plsc

N_NODES = 100000
MEM_DIM = 128
MSG_DIM = 256
B = 4096

_NC = 2
_NS = 16
_NW = _NC * _NS
_CHUNK = B // _NW

_CP_CHUNK = 3200
_CP_LAST = N_NODES - _CP_CHUNK * (_NW - 1)


def _sc_mesh():
    return plsc.VectorSubcoreMesh(
        core_axis_name="c", subcore_axis_name="s", num_cores=_NC, num_subcores=_NS
    )


def _worker_id():
    return lax.axis_index("s") * _NC + lax.axis_index("c")


def _sc_gather(mem, idx):
    @functools.partial(
        pl.kernel,
        out_type=jax.ShapeDtypeStruct((B, MEM_DIM), jnp.float32),
        mesh=_sc_mesh(),
        scratch_types=[
            pltpu.VMEM((_CHUNK,), jnp.int32),
            pltpu.VMEM((_CHUNK, MEM_DIM), jnp.float32),
            pltpu.SemaphoreType.DMA,
        ],
    )
    def gk(mem_hbm, idx_hbm, out_hbm, idx_v, rows_v, sem):
        base = _worker_id() * _CHUNK
        pltpu.sync_copy(idx_hbm.at[pl.ds(base, _CHUNK)], idx_v)
        pltpu.async_copy(mem_hbm.at[idx_v], rows_v, sem).wait()
        pltpu.sync_copy(rows_v, out_hbm.at[pl.ds(base, _CHUNK)])

    return gk(mem, idx)


_GRU_BLK = 1024
_JL_CHUNK = 512
_JL_IBLK = 1024


_CPB = 4000
_NCH = N_NODES // _CPB
_CPS = 6


def _comp_body(x_ref, h_ref, wih_ref, whh_ref, bih_ref, bhh_ref,
               nlane_ref, ncol_ref, mem_hbm, lu_hbm,
               newh_ref, jl_ref, memout_hbm, luout_hbm,
               nb_scratch, *bufs_and_sems):
    bufs = bufs_and_sems[:7]
    lubuf = bufs_and_sems[7]
    isem = bufs_and_sems[8:15]
    osem = bufs_and_sems[15:22]
    lusem_i = bufs_and_sems[22]
    lusem_o = bufs_and_sems[23]
    i = pl.program_id(0)

    def chunk_at(ref, g):
        off = pl.multiple_of(g * _CPB, _CPB)
        return ref.at[pl.ds(off, _CPB)]

    for kk in range(_CPS):
        @pl.when(i > 0)
        def _drain(kk=kk):
            pltpu.make_async_copy(bufs[kk], chunk_at(memout_hbm, 0), osem[kk]).wait()
        g = i * _CPS + kk
        pltpu.make_async_copy(chunk_at(mem_hbm, g), bufs[kk], isem[kk]).start()

    @pl.when(i == B // _GRU_BLK - 1)
    def _extra_in():
        pltpu.make_async_copy(chunk_at(mem_hbm, _NCH - 1), bufs[6], isem[6]).start()

    @pl.when(i == 0)
    def _build_nbcast():
        pltpu.make_async_copy(lu_hbm, lubuf, lusem_i).start()
        nb_scratch[...] = jnp.broadcast_to(ncol_ref[...], (B, 128))

    x = x_ref[...].astype(jnp.bfloat16)
    h32 = h_ref[...]
    h = h32.astype(jnp.bfloat16)
    dn = (((1,), (1,)), ((), ()))
    wih = wih_ref[...].astype(jnp.bfloat16)
    whh = whh_ref[...].astype(jnp.bfloat16)
    gi = lax.dot_general(x, wih, dn, preferred_element_type=jnp.float32) + bih_ref[...]
    gh = lax.dot_general(h, whh, dn, preferred_element_type=jnp.float32) + bhh_ref[...]
    i_r, i_z, i_n = gi[:, :MEM_DIM], gi[:, MEM_DIM : 2 * MEM_DIM], gi[:, 2 * MEM_DIM :]
    h_r, h_z, h_n = gh[:, :MEM_DIM], gh[:, MEM_DIM : 2 * MEM_DIM], gh[:, 2 * MEM_DIM :]
    r = jax.nn.sigmoid(i_r + h_r)
    z = jax.nn.sigmoid(i_z + h_z)
    n = jnp.tanh(i_n + r * h_n)
    newh_ref[...] = n + z * (h32 - n)

    ni = nlane_ref[0]
    nrows = _JL_IBLK // 128
    nchunks = B // _JL_CHUNK

    def _sweep(start):
        def go(ni_op):
            bests = [jnp.full((1, 128), -1, jnp.int32) for _ in range(nrows)]
            for c in range(start, nchunks):
                nj = nb_scratch[pl.ds(c * _JL_CHUNK, _JL_CHUNK), :]
                jv = lax.broadcasted_iota(jnp.int32, (_JL_CHUNK, 128), 0) + c * _JL_CHUNK
                for rr in range(nrows):
                    m = jnp.where(nj == ni_op[rr : rr + 1, :], jv, -1)
                    bests[rr] = jnp.maximum(bests[rr], jnp.max(m, axis=0, keepdims=True))
            return jnp.concatenate(bests, axis=0)
        return go

    nsweep = _JL_IBLK // _JL_CHUNK
    jl_ref[0] = lax.switch(i, [_sweep(s * nsweep) for s in range(B // _JL_IBLK)], ni)

    for kk in range(_CPS):
        g = i * _CPS + kk
        pltpu.make_async_copy(chunk_at(mem_hbm, g), bufs[kk], isem[kk]).wait()
        pltpu.make_async_copy(bufs[kk], chunk_at(memout_hbm, g), osem[kk]).start()

    @pl.when(i == B // _GRU_BLK - 1)
    def _final_drain():
        pltpu.make_async_copy(chunk_at(mem_hbm, _NCH - 1), bufs[6], isem[6]).wait()
        pltpu.make_async_copy(bufs[6], chunk_at(memout_hbm, _NCH - 1), osem[6]).start()
        pltpu.make_async_copy(lu_hbm, lubuf, lusem_i).wait()
        pltpu.make_async_copy(lubuf, luout_hbm, lusem_o).start()
        for kk in range(_CPS):
            pltpu.make_async_copy(bufs[kk], chunk_at(memout_hbm, 0), osem[kk]).wait()
        pltpu.make_async_copy(bufs[6], chunk_at(memout_hbm, 0), osem[6]).wait()
        pltpu.make_async_copy(lubuf, luout_hbm, lusem_o).wait()


def _tc_compute(x, h, W_ih, W_hh, b_ih, b_hh, nodes, mem, lu):
    bih = b_ih.reshape(1, -1)
    bhh = b_hh.reshape(1, -1)
    nlane = nodes.reshape(B // _JL_IBLK, _JL_IBLK // 128, 128)
    ncol = nodes.reshape(B, 1)
    blk = lambda i: (i, 0)
    blk3 = lambda i: (i, 0, 0)
    const2 = lambda i: (0, 0)
    new_h, jl, mem_out, lu_out = pl.pallas_call(
        _comp_body,
        grid=(B // _GRU_BLK,),
        in_specs=[
            pl.BlockSpec((_GRU_BLK, MSG_DIM), blk),
            pl.BlockSpec((_GRU_BLK, MEM_DIM), blk),
            pl.BlockSpec((3 * MEM_DIM, MSG_DIM), const2),
            pl.BlockSpec((3 * MEM_DIM, MEM_DIM), const2),
            pl.BlockSpec((1, 3 * MEM_DIM), const2),
            pl.BlockSpec((1, 3 * MEM_DIM), const2),
            pl.BlockSpec((1, _JL_IBLK // 128, 128), blk3),
            pl.BlockSpec((B, 1), const2),
            pl.BlockSpec(memory_space=pltpu.HBM),
            pl.BlockSpec(memory_space=pltpu.HBM),
        ],
        out_specs=[
            pl.BlockSpec((_GRU_BLK, MEM_DIM), blk),
            pl.BlockSpec((1, _JL_IBLK // 128, 128), blk3),
            pl.BlockSpec(memory_space=pltpu.HBM),
            pl.BlockSpec(memory_space=pltpu.HBM),
        ],
        out_shape=[
            jax.ShapeDtypeStruct((B, MEM_DIM), jnp.float32),
            jax.ShapeDtypeStruct((B // _JL_IBLK, _JL_IBLK // 128, 128), jnp.int32),
            jax.ShapeDtypeStruct((N_NODES, MEM_DIM), jnp.float32),
            jax.ShapeDtypeStruct((N_NODES,), jnp.float32),
        ],
        scratch_shapes=[pltpu.VMEM((B, 128), jnp.int32)]
        + [pltpu.VMEM((_CPB, MEM_DIM), jnp.float32) for _ in range(7)]
        + [pltpu.VMEM((N_NODES,), jnp.float32)]
        + [pltpu.SemaphoreType.DMA for _ in range(16)],
    )(x, h, W_ih, W_hh, bih, bhh, nlane, ncol, mem, lu)
    return new_h, jl, mem_out, lu_out


def _sc_scatter(new_h, j_last, idx, ts, mem_ref, lu_ref):
    @functools.partial(
        pl.kernel,
        out_type=(),
        mesh=_sc_mesh(),
        scratch_types=[
            pltpu.VMEM((_CHUNK // 2,), jnp.int32),
            pltpu.VMEM((_CHUNK // 2,), jnp.int32),
            pltpu.VMEM((_CHUNK // 2,), jnp.int32),
            pltpu.VMEM((_CHUNK // 2,), jnp.int32),
            pltpu.VMEM((_CHUNK // 2, MEM_DIM), jnp.float32),
            pltpu.VMEM((_CHUNK // 2, MEM_DIM), jnp.float32),
            pltpu.VMEM((_CHUNK,), jnp.float32),
            pltpu.SemaphoreType.DMA,
            pltpu.SemaphoreType.DMA,
            pltpu.SemaphoreType.DMA,
        ],
    )
    def sk(newh_hbm, jl_hbm, idx_hbm, ts_hbm, outmem_hbm, outlu_hbm,
           jla_v, jlb_v, idxa_v, idxb_v, rowsa_v, rowsb_v, ts_v,
           sema, semb, semt):
        w = _worker_id()
        base = w * _CHUNK
        half = _CHUNK // 2
        pltpu.sync_copy(jl_hbm.at[w // 8, w % 8, pl.ds(0, half)], jla_v)
        pltpu.sync_copy(jl_hbm.at[w // 8, w % 8, pl.ds(half, half)], jlb_v)
        pltpu.sync_copy(idx_hbm.at[pl.ds(base, half)], idxa_v)
        pltpu.sync_copy(idx_hbm.at[pl.ds(base + half, half)], idxb_v)
        ga = pltpu.async_copy(newh_hbm.at[jla_v], rowsa_v, sema)
        gb = pltpu.async_copy(newh_hbm.at[jlb_v], rowsb_v, semb)
        gt = pltpu.async_copy(ts_hbm.at[jla_v], ts_v.at[pl.ds(0, half)], semt)
        ga.wait()
        sa = pltpu.async_copy(rowsa_v, outmem_hbm.at[idxa_v], sema)
        gb.wait()
        sb = pltpu.async_copy(rowsb_v, outmem_hbm.at[idxb_v], semb)
        gt.wait()
        gt2 = pltpu.async_copy(ts_hbm.at[jlb_v], ts_v.at[pl.ds(half, half)], semt)
        gt2.wait()
        st1 = pltpu.async_copy(ts_v.at[pl.ds(0, half)], outlu_hbm.at[idxa_v], semt)
        st1.wait()
        st2 = pltpu.async_copy(ts_v.at[pl.ds(half, half)], outlu_hbm.at[idxb_v], semt)
        sa.wait()
        sb.wait()
        st2.wait()

    sk(new_h, j_last, idx, ts, mem_ref, lu_ref)


def kernel(memory_tensor, last_update, unique_nodes, unique_messages, unique_ts, W_ih, W_hh, b_ih, b_hh):
    h = _sc_gather(memory_tensor, unique_nodes)
    new_h, j_last, mem_out, lu_out = _tc_compute(
        unique_messages, h, W_ih, W_hh, b_ih, b_hh, unique_nodes, memory_tensor, last_update
    )
    mem_ref = jax.new_ref(mem_out)
    lu_ref = jax.new_ref(lu_out)
    _sc_scatter(new_h, j_last, unique_nodes, unique_ts, mem_ref, lu_ref)
    return mem_ref[...], lu_ref[...]
